# baseline (speedup 1.0000x reference)
import jax
import jax.numpy as jnp
from jax import lax
from jax.experimental import pallas as pl
from jax.experimental.pallas import tpu as pltpu
from jax.experimental.pallas import tpu_sc as plsc

B, T, L, D, V = 4096, 20, 20, 128, 30000
K = T * L
NC, NS = 2, 16
NW = NC * NS
BPW = B // NW
CH = 80
NCH = K // CH
ROWS_PW = BPW * NCH
TOTAL = ROWS_PW
NBUF = 4
UU = 4
PAIR = UU * NCH
RU = 4
PVT = 938
PBLK = 64
PNB = -(-PVT // PBLK)


def _pack_body(tab_hbm, pk_hbm, fbuf, obuf):
    wid = lax.axis_index("s") * NC + lax.axis_index("c")
    base = wid * PVT
    hround = jnp.full((16,), 0x8000, jnp.uint32)
    himask = jnp.full((16,), 0xFFFF0000, jnp.uint32)

    def blk(bi, carry):
        s = jnp.minimum(base + bi * PBLK, V - PBLK)
        pltpu.sync_copy(tab_hbm.at[pl.ds(s, PBLK)], fbuf)

        def rb(r, c2):
            for c in range(4):
                lo = plsc.bitcast(fbuf[r, pl.ds(16 * c, 16)], jnp.uint32)
                hi = plsc.bitcast(fbuf[r, pl.ds(64 + 16 * c, 16)], jnp.uint32)
                w = ((lo + hround) >> 16) | ((hi + hround) & himask)
                obuf[r, pl.ds(16 * c, 16)] = plsc.bitcast(w, jnp.int32)
            return c2

        lax.fori_loop(0, PBLK, rb, 0)
        pltpu.sync_copy(obuf, pk_hbm.at[pl.ds(s, PBLK)])
        return carry

    lax.fori_loop(0, PNB, blk, 0)


def _encoder_body(idx_hbm, tab_hbm, out_hbm, idx_v, rows0, rows1, rows2,
                  rows3, out_v, sem0, sem1, sem2, sem3):
    wid = lax.axis_index("s") * NC + lax.axis_index("c")
    pltpu.sync_copy(idx_hbm.at[pl.ds(wid * BPW, BPW)], idx_v)
    bufs = (rows0, rows1, rows2, rows3)
    sems = (sem0, sem1, sem2, sem3)
    zero16 = tuple(jnp.zeros((16,), jnp.float32) for _ in range(16))
    scale = jnp.float32(1.0 / K)

    def _idx_ref(g):
        return idx_v.at[g // NCH, pl.ds(CH * (g % NCH), CH)]

    def start(g, p):
        pltpu.async_copy(tab_hbm.at[_idx_ref(g)], bufs[p], sems[p])

    def wait(g, p):
        pltpu.make_async_copy(tab_hbm.at[_idx_ref(g)], bufs[p], sems[p]).wait()

    def reduce_chunk(buf, accs):
        def rb(j, a):
            a = list(a)
            for u in range(RU):
                for c in range(4):
                    x = buf[RU * j + u, pl.ds(16 * c, 16)]
                    lo = plsc.bitcast(x << 16, jnp.float32)
                    hi = plsc.bitcast(x, jnp.float32)
                    a[c] = a[c] + lo
                    a[4 + c] = a[4 + c] + hi
            return tuple(a)

        return lax.fori_loop(0, CH // RU, rb, accs)

    for g0 in range(NBUF - 1):
        start(g0, g0)

    def step_body(p, carry):
        accs = zero16
        for q in range(PAIR):
            g = p * PAIR + q
            wait(g, q % NBUF)

            @pl.when(g + NBUF - 1 < TOTAL)
            def _():
                start(g + NBUF - 1, (q + NBUF - 1) % NBUF)

            accs = reduce_chunk(bufs[q % NBUF], accs)
            if q % NCH == NCH - 1:
                b = UU * p + q // NCH
                for c in range(4):
                    out_v[b, pl.ds(16 * c, 16)] = accs[c] * scale
                    out_v[b, pl.ds(64 + 16 * c, 16)] = accs[4 + c] * scale
                accs = zero16
        return carry

    lax.fori_loop(0, BPW // UU, step_body, 0)
    pltpu.sync_copy(out_v, out_hbm.at[pl.ds(wid * BPW, BPW)])


def kernel(word_ids, table):
    idx = word_ids.reshape(B, K)
    mesh = plsc.VectorSubcoreMesh(core_axis_name="c", subcore_axis_name="s")
    packf = pl.kernel(
        _pack_body,
        mesh=mesh,
        compiler_params=pltpu.CompilerParams(
            needs_layout_passes=False, use_tc_tiling_on_sc=False),
        out_type=jax.ShapeDtypeStruct((V, D // 2), jnp.int32),
        scratch_types=[
            pltpu.VMEM((PBLK, D), jnp.float32),
            pltpu.VMEM((PBLK, D // 2), jnp.int32),
        ],
    )
    tab_pk = packf(table)
    f = pl.kernel(
        _encoder_body,
        mesh=mesh,
        compiler_params=pltpu.CompilerParams(
            needs_layout_passes=False, use_tc_tiling_on_sc=False),
        out_type=jax.ShapeDtypeStruct((B, D), jnp.float32),
        scratch_types=[
            pltpu.VMEM((BPW, K), jnp.int32),
            pltpu.VMEM((CH, D // 2), jnp.int32),
            pltpu.VMEM((CH, D // 2), jnp.int32),
            pltpu.VMEM((CH, D // 2), jnp.int32),
            pltpu.VMEM((CH, D // 2), jnp.int32),
            pltpu.VMEM((BPW, D), jnp.float32),
            pltpu.SemaphoreType.DMA,
            pltpu.SemaphoreType.DMA,
            pltpu.SemaphoreType.DMA,
            pltpu.SemaphoreType.DMA,
        ],
    )
    return f(idx, tab_pk)


# speedup vs baseline: 1.2766x; 1.2766x over previous
import jax
import jax.numpy as jnp
from jax import lax
from jax.experimental import pallas as pl
from jax.experimental.pallas import tpu as pltpu
from jax.experimental.pallas import tpu_sc as plsc

B, T, L, D, V = 4096, 20, 20, 128, 30000
K = T * L
NC, NS = 2, 16
NW = NC * NS
BPW = B // NW
CH = 80
NCH = K // CH
ROWS_PW = BPW * NCH
TOTAL = ROWS_PW
NBUF = 8
UU = 8
PAIR = UU * NCH
RU = 4


def _encoder_body(idx_hbm, tab_hbm, out_hbm, idx_v, rows0, rows1, rows2,
                  rows3, rows4, rows5, rows6, rows7, out_v, sem0, sem1, sem2,
                  sem3, sem4, sem5, sem6, sem7):
    wid = lax.axis_index("s") * NC + lax.axis_index("c")
    pltpu.sync_copy(idx_hbm.at[pl.ds(wid * BPW, BPW)], idx_v)
    bufs = (rows0, rows1, rows2, rows3, rows4, rows5, rows6, rows7)
    sems = (sem0, sem1, sem2, sem3, sem4, sem5, sem6, sem7)
    zero16 = tuple(jnp.zeros((16,), jnp.float32) for _ in range(16))
    scale = jnp.float32(1.0 / K)

    def _idx_ref(g):
        return idx_v.at[g // NCH, pl.ds(CH * (g % NCH), CH)]

    def start(g, p):
        pltpu.async_copy(tab_hbm.at[_idx_ref(g)], bufs[p], sems[p])

    def wait(g, p):
        pltpu.make_async_copy(tab_hbm.at[_idx_ref(g)], bufs[p], sems[p]).wait()

    def reduce_chunk(buf, accs):
        def rb(j, a):
            a = list(a)
            for u in range(RU):
                for c in range(4):
                    x = buf[RU * j + u, pl.ds(16 * c, 16)]
                    lo = plsc.bitcast(x << 16, jnp.float32)
                    hi = plsc.bitcast(x, jnp.float32)
                    a[c] = a[c] + lo
                    a[4 + c] = a[4 + c] + hi
            return tuple(a)

        return lax.fori_loop(0, CH // RU, rb, accs)

    for g0 in range(NBUF - 1):
        start(g0, g0)

    def step_body(p, carry):
        accs = zero16
        for q in range(PAIR):
            g = p * PAIR + q
            wait(g, q % NBUF)

            @pl.when(g + NBUF - 1 < TOTAL)
            def _():
                start(g + NBUF - 1, (q + NBUF - 1) % NBUF)

            accs = reduce_chunk(bufs[q % NBUF], accs)
            if q % NCH == NCH - 1:
                b = UU * p + q // NCH
                for c in range(4):
                    out_v[b, pl.ds(16 * c, 16)] = accs[c] * scale
                    out_v[b, pl.ds(64 + 16 * c, 16)] = accs[4 + c] * scale
                accs = zero16
        return carry

    lax.fori_loop(0, BPW // UU, step_body, 0)
    pltpu.sync_copy(out_v, out_hbm.at[pl.ds(wid * BPW, BPW)])


def kernel(word_ids, table):
    idx = word_ids.reshape(B, K)
    bu = lax.bitcast_convert_type(table, jnp.uint32)
    half = jnp.uint32(0x8000)
    pk = (((bu[:, : D // 2] + half) >> 16)
          | ((bu[:, D // 2:] + half) & jnp.uint32(0xFFFF0000)))
    tab_pk = lax.bitcast_convert_type(pk, jnp.int32)
    mesh = plsc.VectorSubcoreMesh(core_axis_name="c", subcore_axis_name="s")
    f = pl.kernel(
        _encoder_body,
        mesh=mesh,
        compiler_params=pltpu.CompilerParams(
            needs_layout_passes=False, use_tc_tiling_on_sc=False),
        out_type=jax.ShapeDtypeStruct((B, D), jnp.float32),
        scratch_types=[
            pltpu.VMEM((BPW, K), jnp.int32),
            pltpu.VMEM((CH, D // 2), jnp.int32),
            pltpu.VMEM((CH, D // 2), jnp.int32),
            pltpu.VMEM((CH, D // 2), jnp.int32),
            pltpu.VMEM((CH, D // 2), jnp.int32),
            pltpu.VMEM((CH, D // 2), jnp.int32),
            pltpu.VMEM((CH, D // 2), jnp.int32),
            pltpu.VMEM((CH, D // 2), jnp.int32),
            pltpu.VMEM((CH, D // 2), jnp.int32),
            pltpu.VMEM((BPW, D), jnp.float32),
            pltpu.SemaphoreType.DMA,
            pltpu.SemaphoreType.DMA,
            pltpu.SemaphoreType.DMA,
            pltpu.SemaphoreType.DMA,
            pltpu.SemaphoreType.DMA,
            pltpu.SemaphoreType.DMA,
            pltpu.SemaphoreType.DMA,
            pltpu.SemaphoreType.DMA,
        ],
    )
    return f(idx, tab_pk)
